# Initial kernel scaffold; baseline (speedup 1.0000x reference)
import functools

import jax
import jax.numpy as jnp
from jax import lax
from jax.experimental import pallas as pl
from jax.experimental.pallas import tpu as pltpu
from jax.experimental.pallas import tpu_sc as plsc

N = 10000
E = 320000
D = 128
NC = 2
NS = 16
NW = NC * NS

CHUNK = 128
NCHUNKS = E // CHUNK
BASE_CH = NCHUNKS // NW
EXTRA = NCHUNKS - BASE_CH * NW
ROWS_PER_SUB = N // NS
USERS_PER_W = 2048 // NW

_mesh = plsc.VectorSubcoreMesh(core_axis_name="c", subcore_axis_name="s")


def _worker_id():
    return lax.axis_index("s") * NC + lax.axis_index("c")


@functools.partial(
    pl.kernel,
    out_type=(
        jax.ShapeDtypeStruct((NC, N, 16), jnp.float32),
        jax.ShapeDtypeStruct((NC, N, 16), jnp.float32),
    ),
    mesh=_mesh,
    scratch_types=[
        pltpu.VMEM_SHARED((N, 16), jnp.float32),
        pltpu.VMEM_SHARED((N, 16), jnp.float32),
        pltpu.VMEM((CHUNK,), jnp.int32),
        pltpu.VMEM((CHUNK,), jnp.int32),
        pltpu.VMEM((CHUNK, 16), jnp.float32),
    ],
)
def _hist_kernel(src_hbm, dst_hbm, zeros16_hbm, ones_hbm,
                 degs_out, degd_out, acc_s, acc_d, sidx, didx, ones_v):
    c = lax.axis_index("c")
    s = lax.axis_index("s")
    w = _worker_id()
    r0 = s * ROWS_PER_SUB
    pltpu.sync_copy(zeros16_hbm.at[pl.ds(r0, ROWS_PER_SUB)],
                    acc_s.at[pl.ds(r0, ROWS_PER_SUB)])
    pltpu.sync_copy(zeros16_hbm.at[pl.ds(r0, ROWS_PER_SUB)],
                    acc_d.at[pl.ds(r0, ROWS_PER_SUB)])
    pltpu.sync_copy(ones_hbm, ones_v)
    plsc.subcore_barrier()

    n_my = BASE_CH + (w < EXTRA).astype(jnp.int32)

    @pl.loop(0, n_my)
    def _(i):
        e0 = (w + i * NW) * CHUNK
        pltpu.sync_copy(src_hbm.at[pl.ds(e0, CHUNK)], sidx)
        pltpu.sync_copy(dst_hbm.at[pl.ds(e0, CHUNK)], didx)
        pltpu.sync_copy(ones_v, acc_s.at[sidx], add=True)
        pltpu.sync_copy(ones_v, acc_d.at[didx], add=True)

    plsc.subcore_barrier()
    pltpu.sync_copy(acc_s.at[pl.ds(r0, ROWS_PER_SUB)],
                    degs_out.at[c, pl.ds(r0, ROWS_PER_SUB)])
    pltpu.sync_copy(acc_d.at[pl.ds(r0, ROWS_PER_SUB)],
                    degd_out.at[c, pl.ds(r0, ROWS_PER_SUB)])


@functools.partial(
    pl.kernel,
    out_type=jax.ShapeDtypeStruct((NC, N, D), jnp.float32),
    mesh=_mesh,
    scratch_types=[
        pltpu.VMEM_SHARED((N, D), jnp.float32),
        pltpu.VMEM((CHUNK,), jnp.int32),
        pltpu.VMEM((CHUNK,), jnp.int32),
        pltpu.VMEM((CHUNK, D), jnp.float32),
        pltpu.SemaphoreType.DMA,
    ],
)
def _scatter_kernel(table_hbm, src_hbm, dst_hbm, zerosd_hbm,
                    out_hbm, acc, sidx, didx, rows, sem):
    c = lax.axis_index("c")
    s = lax.axis_index("s")
    w = _worker_id()
    r0 = s * ROWS_PER_SUB
    pltpu.sync_copy(zerosd_hbm.at[pl.ds(r0, ROWS_PER_SUB)],
                    acc.at[pl.ds(r0, ROWS_PER_SUB)])
    plsc.subcore_barrier()

    n_my = BASE_CH + (w < EXTRA).astype(jnp.int32)

    @pl.loop(0, n_my)
    def _(i):
        e0 = (w + i * NW) * CHUNK
        pltpu.sync_copy(src_hbm.at[pl.ds(e0, CHUNK)], sidx)
        pltpu.sync_copy(dst_hbm.at[pl.ds(e0, CHUNK)], didx)
        pltpu.async_copy(table_hbm.at[sidx], rows, sem).wait()
        pltpu.sync_copy(rows, acc.at[didx], add=True)

    plsc.subcore_barrier()
    pltpu.sync_copy(acc.at[pl.ds(r0, ROWS_PER_SUB)],
                    out_hbm.at[c, pl.ds(r0, ROWS_PER_SUB)])


@functools.partial(
    pl.kernel,
    out_type=jax.ShapeDtypeStruct((2048, D), jnp.float32),
    mesh=_mesh,
    scratch_types=[
        pltpu.VMEM((USERS_PER_W,), jnp.int32),
        pltpu.VMEM((USERS_PER_W, D), jnp.float32),
        pltpu.SemaphoreType.DMA,
    ],
)
def _user_gather_kernel(h_hbm, users_hbm, out_hbm, uidx, rows, sem):
    w = _worker_id()
    base = w * USERS_PER_W
    pltpu.sync_copy(users_hbm.at[pl.ds(base, USERS_PER_W)], uidx)
    pltpu.async_copy(h_hbm.at[uidx], rows, sem).wait()
    pltpu.sync_copy(rows, out_hbm.at[pl.ds(base, USERS_PER_W)])


_BLK = 1000


def _mm_body(x_ref, w_ref, o_ref):
    o_ref[...] = jnp.dot(x_ref[...], w_ref[...],
                         preferred_element_type=jnp.float32)


def _mm(x, w):
    n = x.shape[0]
    return pl.pallas_call(
        _mm_body,
        grid=(n // _BLK,),
        in_specs=[
            pl.BlockSpec((_BLK, x.shape[1]), lambda i: (i, 0)),
            pl.BlockSpec(w.shape, lambda i: (0, 0)),
        ],
        out_specs=pl.BlockSpec((_BLK, w.shape[1]), lambda i: (i, 0)),
        out_shape=jax.ShapeDtypeStruct((n, w.shape[1]), jnp.float32),
    )(x, w)


def _norm_from(deg_ref):
    d = deg_ref[0, :, 0:1] + deg_ref[1, :, 0:1]
    return lax.rsqrt(jnp.maximum(d, 1.0))


def _scale_body(hw_ref, degs_ref, o_ref):
    o_ref[...] = hw_ref[...] * _norm_from(degs_ref)


def _scale(hw, degs_p):
    return pl.pallas_call(
        _scale_body,
        grid=(N // _BLK,),
        in_specs=[
            pl.BlockSpec((_BLK, D), lambda i: (i, 0)),
            pl.BlockSpec((NC, _BLK, 16), lambda i: (0, i, 0)),
        ],
        out_specs=pl.BlockSpec((_BLK, D), lambda i: (i, 0)),
        out_shape=jax.ShapeDtypeStruct((N, D), jnp.float32),
    )(hw, degs_p)


def _layer_mm_body(p_ref, degd_ref, degs_ref, b_ref, w_ref, o_ref):
    agg = p_ref[0] + p_ref[1]
    h = jax.nn.relu(agg * _norm_from(degd_ref) + b_ref[...])
    o_ref[...] = jnp.dot(h, w_ref[...],
                         preferred_element_type=jnp.float32) * _norm_from(degs_ref)


def _layer_mm(p, degd_p, degs_p, b, w):
    return pl.pallas_call(
        _layer_mm_body,
        grid=(N // _BLK,),
        in_specs=[
            pl.BlockSpec((NC, _BLK, D), lambda i: (0, i, 0)),
            pl.BlockSpec((NC, _BLK, 16), lambda i: (0, i, 0)),
            pl.BlockSpec((NC, _BLK, 16), lambda i: (0, i, 0)),
            pl.BlockSpec((1, D), lambda i: (0, 0)),
            pl.BlockSpec((D, D), lambda i: (0, 0)),
        ],
        out_specs=pl.BlockSpec((_BLK, D), lambda i: (i, 0)),
        out_shape=jax.ShapeDtypeStruct((N, D), jnp.float32),
    )(p, degd_p, degs_p, b, w)


def _layer_out_body(p_ref, degd_ref, b_ref, o_ref):
    agg = p_ref[0] + p_ref[1]
    o_ref[...] = jax.nn.relu(agg * _norm_from(degd_ref) + b_ref[...])


def _layer_out(p, degd_p, b):
    return pl.pallas_call(
        _layer_out_body,
        grid=(N // _BLK,),
        in_specs=[
            pl.BlockSpec((NC, _BLK, D), lambda i: (0, i, 0)),
            pl.BlockSpec((NC, _BLK, 16), lambda i: (0, i, 0)),
            pl.BlockSpec((1, D), lambda i: (0, 0)),
        ],
        out_specs=pl.BlockSpec((_BLK, D), lambda i: (i, 0)),
        out_shape=jax.ShapeDtypeStruct((N, D), jnp.float32),
    )(p, degd_p, b)


def _mlp_body(uh_ref, w1_ref, b1_ref, w2_ref, b2_ref, o_ref):
    t = jnp.tanh(jnp.dot(uh_ref[...], w1_ref[...],
                         preferred_element_type=jnp.float32) + b1_ref[...])
    o_ref[...] = jnp.dot(t, w2_ref[...],
                         preferred_element_type=jnp.float32) + b2_ref[...]


def _mlp(uh, w1, b1, w2, b2):
    return pl.pallas_call(
        _mlp_body,
        out_shape=jax.ShapeDtypeStruct((uh.shape[0], w2.shape[1]), jnp.float32),
    )(uh, w1, b1, w2, b2)


def kernel(features, W1, b1, W2, b2, Ws1, bs1, Ws2, bs2, edge_index, users):
    src = edge_index[0].astype(jnp.int32)
    dst = edge_index[1].astype(jnp.int32)
    users = users.astype(jnp.int32)
    zeros16 = jnp.zeros((N, 16), jnp.float32)
    zerosd = jnp.zeros((N, D), jnp.float32)
    ones16 = jnp.ones((CHUNK, 16), jnp.float32)

    degs_p, degd_p = _hist_kernel(src, dst, zeros16, ones16)
    hw1 = _mm(features, W1)
    scaled1 = _scale(hw1, degs_p)
    p1 = _scatter_kernel(scaled1, src, dst, zerosd)
    scaled2 = _layer_mm(p1, degd_p, degs_p, b1.reshape(1, D), W2)
    p2 = _scatter_kernel(scaled2, src, dst, zerosd)
    h = _layer_out(p2, degd_p, b2.reshape(1, D))
    uh = _user_gather_kernel(h, users)
    R = _mlp(uh, Ws1, bs1.reshape(1, -1), Ws2, bs2.reshape(1, -1))
    return (R, h)


# speedup vs baseline: 8.1703x; 8.1703x over previous
import functools

import jax
import jax.numpy as jnp
from jax import lax
from jax.experimental import pallas as pl
from jax.experimental.pallas import tpu as pltpu
from jax.experimental.pallas import tpu_sc as plsc

N = 10000
E = 320000
D = 128
NC = 2
NS = 16
NW = NC * NS

CHUNK = 128
NCHUNKS = E // CHUNK
BASE_CH = NCHUNKS // NW
EXTRA = NCHUNKS - BASE_CH * NW
STRIPE = 624
REM_BASE = NS * STRIPE
REM = N - REM_BASE
USERS_PER_W = 2048 // NW

_mesh = plsc.VectorSubcoreMesh(core_axis_name="c", subcore_axis_name="s")


def _worker_id():
    return lax.axis_index("s") * NC + lax.axis_index("c")


def _striped(s, fn):
    fn(s * STRIPE, STRIPE)

    @pl.when(s == NS - 1)
    def _():
        fn(REM_BASE, REM)


@functools.partial(
    pl.kernel,
    out_type=jax.ShapeDtypeStruct((NC, N, D), jnp.float32),
    mesh=_mesh,
    scratch_types=[
        pltpu.VMEM_SHARED((N, D), jnp.float32),
        pltpu.VMEM((CHUNK,), jnp.int32),
        pltpu.VMEM((CHUNK,), jnp.int32),
        pltpu.VMEM((CHUNK, D), jnp.float32),
        pltpu.VMEM((CHUNK, D), jnp.float32),
    ],
)
def _hist_kernel(src_hbm, dst_hbm, zerosd_hbm, ones_a_hbm, ones_b_hbm,
                 deg_out, acc, sidx, didx, ones_a, ones_b):
    c = lax.axis_index("c")
    s = lax.axis_index("s")
    w = _worker_id()

    def _zero(r0, nr):
        pltpu.sync_copy(zerosd_hbm.at[pl.ds(r0, nr)], acc.at[pl.ds(r0, nr)])

    _striped(s, _zero)
    pltpu.sync_copy(ones_a_hbm, ones_a)
    pltpu.sync_copy(ones_b_hbm, ones_b)
    plsc.subcore_barrier()

    n_my = BASE_CH + (w < EXTRA).astype(jnp.int32)

    @pl.loop(0, n_my)
    def _(i):
        e0 = (w + i * NW) * CHUNK
        pltpu.sync_copy(src_hbm.at[pl.ds(e0, CHUNK)], sidx)
        pltpu.sync_copy(dst_hbm.at[pl.ds(e0, CHUNK)], didx)
        pltpu.sync_copy(ones_a, acc.at[sidx], add=True)
        pltpu.sync_copy(ones_b, acc.at[didx], add=True)

    plsc.subcore_barrier()

    def _wout(r0, nr):
        pltpu.sync_copy(acc.at[pl.ds(r0, nr)], deg_out.at[c, pl.ds(r0, nr)])

    _striped(s, _wout)


@functools.partial(
    pl.kernel,
    out_type=jax.ShapeDtypeStruct((NC, N, D), jnp.float32),
    mesh=_mesh,
    scratch_types=[
        pltpu.VMEM_SHARED((N, D), jnp.float32),
        pltpu.VMEM((CHUNK,), jnp.int32),
        pltpu.VMEM((CHUNK,), jnp.int32),
        pltpu.VMEM((CHUNK, D), jnp.float32),
        pltpu.SemaphoreType.DMA,
    ],
)
def _scatter_kernel(table_hbm, src_hbm, dst_hbm, zerosd_hbm,
                    out_hbm, acc, sidx, didx, rows, sem):
    c = lax.axis_index("c")
    s = lax.axis_index("s")
    w = _worker_id()

    def _zero(r0, nr):
        pltpu.sync_copy(zerosd_hbm.at[pl.ds(r0, nr)], acc.at[pl.ds(r0, nr)])

    _striped(s, _zero)
    plsc.subcore_barrier()

    n_my = BASE_CH + (w < EXTRA).astype(jnp.int32)

    @pl.loop(0, n_my)
    def _(i):
        e0 = (w + i * NW) * CHUNK
        pltpu.sync_copy(src_hbm.at[pl.ds(e0, CHUNK)], sidx)
        pltpu.sync_copy(dst_hbm.at[pl.ds(e0, CHUNK)], didx)
        pltpu.async_copy(table_hbm.at[sidx], rows, sem).wait()
        pltpu.sync_copy(rows, acc.at[didx], add=True)

    plsc.subcore_barrier()

    def _wout(r0, nr):
        pltpu.sync_copy(acc.at[pl.ds(r0, nr)], out_hbm.at[c, pl.ds(r0, nr)])

    _striped(s, _wout)


@functools.partial(
    pl.kernel,
    out_type=jax.ShapeDtypeStruct((2048, D), jnp.float32),
    mesh=_mesh,
    scratch_types=[
        pltpu.VMEM((USERS_PER_W,), jnp.int32),
        pltpu.VMEM((USERS_PER_W, D), jnp.float32),
        pltpu.SemaphoreType.DMA,
    ],
)
def _user_gather_kernel(h_hbm, users_hbm, out_hbm, uidx, rows, sem):
    w = _worker_id()
    base = w * USERS_PER_W
    pltpu.sync_copy(users_hbm.at[pl.ds(base, USERS_PER_W)], uidx)
    pltpu.async_copy(h_hbm.at[uidx], rows, sem).wait()
    pltpu.sync_copy(rows, out_hbm.at[pl.ds(base, USERS_PER_W)])


_BLK = 1000


def _mm_body(x_ref, w_ref, o_ref):
    o_ref[...] = jnp.dot(x_ref[...], w_ref[...],
                         preferred_element_type=jnp.float32)


def _mm(x, w):
    n = x.shape[0]
    return pl.pallas_call(
        _mm_body,
        grid=(n // _BLK,),
        in_specs=[
            pl.BlockSpec((_BLK, x.shape[1]), lambda i: (i, 0)),
            pl.BlockSpec(w.shape, lambda i: (0, 0)),
        ],
        out_specs=pl.BlockSpec((_BLK, w.shape[1]), lambda i: (i, 0)),
        out_shape=jax.ShapeDtypeStruct((n, w.shape[1]), jnp.float32),
    )(x, w)


def _norm_from(deg_ref, col):
    d = deg_ref[0, :, col:col + 1] + deg_ref[1, :, col:col + 1]
    return lax.rsqrt(jnp.maximum(d, 1.0))


def _scale_body(hw_ref, deg_ref, o_ref):
    o_ref[...] = hw_ref[...] * _norm_from(deg_ref, 0)


def _scale(hw, deg_p):
    return pl.pallas_call(
        _scale_body,
        grid=(N // _BLK,),
        in_specs=[
            pl.BlockSpec((_BLK, D), lambda i: (i, 0)),
            pl.BlockSpec((NC, _BLK, D), lambda i: (0, i, 0)),
        ],
        out_specs=pl.BlockSpec((_BLK, D), lambda i: (i, 0)),
        out_shape=jax.ShapeDtypeStruct((N, D), jnp.float32),
    )(hw, deg_p)


def _layer_mm_body(p_ref, deg_ref, b_ref, w_ref, o_ref):
    agg = p_ref[0] + p_ref[1]
    h = jax.nn.relu(agg * _norm_from(deg_ref, 1) + b_ref[...])
    o_ref[...] = jnp.dot(h, w_ref[...],
                         preferred_element_type=jnp.float32) * _norm_from(deg_ref, 0)


def _layer_mm(p, deg_p, b, w):
    return pl.pallas_call(
        _layer_mm_body,
        grid=(N // _BLK,),
        in_specs=[
            pl.BlockSpec((NC, _BLK, D), lambda i: (0, i, 0)),
            pl.BlockSpec((NC, _BLK, D), lambda i: (0, i, 0)),
            pl.BlockSpec((1, D), lambda i: (0, 0)),
            pl.BlockSpec((D, D), lambda i: (0, 0)),
        ],
        out_specs=pl.BlockSpec((_BLK, D), lambda i: (i, 0)),
        out_shape=jax.ShapeDtypeStruct((N, D), jnp.float32),
    )(p, deg_p, b, w)


def _layer_out_body(p_ref, deg_ref, b_ref, o_ref):
    agg = p_ref[0] + p_ref[1]
    o_ref[...] = jax.nn.relu(agg * _norm_from(deg_ref, 1) + b_ref[...])


def _layer_out(p, deg_p, b):
    return pl.pallas_call(
        _layer_out_body,
        grid=(N // _BLK,),
        in_specs=[
            pl.BlockSpec((NC, _BLK, D), lambda i: (0, i, 0)),
            pl.BlockSpec((NC, _BLK, D), lambda i: (0, i, 0)),
            pl.BlockSpec((1, D), lambda i: (0, 0)),
        ],
        out_specs=pl.BlockSpec((_BLK, D), lambda i: (i, 0)),
        out_shape=jax.ShapeDtypeStruct((N, D), jnp.float32),
    )(p, deg_p, b)


def _mlp_body(uh_ref, w1_ref, b1_ref, w2_ref, b2_ref, o_ref):
    t = jnp.tanh(jnp.dot(uh_ref[...], w1_ref[...],
                         preferred_element_type=jnp.float32) + b1_ref[...])
    o_ref[...] = jnp.dot(t, w2_ref[...],
                         preferred_element_type=jnp.float32) + b2_ref[...]


def _mlp(uh, w1, b1, w2, b2):
    return pl.pallas_call(
        _mlp_body,
        out_shape=jax.ShapeDtypeStruct((uh.shape[0], w2.shape[1]), jnp.float32),
    )(uh, w1, b1, w2, b2)


def kernel(features, W1, b1, W2, b2, Ws1, bs1, Ws2, bs2, edge_index, users):
    src = edge_index[0].astype(jnp.int32)
    dst = edge_index[1].astype(jnp.int32)
    users = users.astype(jnp.int32)
    zerosd = jnp.zeros((N, D), jnp.float32)
    col = jnp.arange(D, dtype=jnp.int32)[None, :]
    ones_a = jnp.broadcast_to((col == 0).astype(jnp.float32), (CHUNK, D))
    ones_b = jnp.broadcast_to((col == 1).astype(jnp.float32), (CHUNK, D))

    deg_p = _hist_kernel(src, dst, zerosd, ones_a, ones_b)
    hw1 = _mm(features, W1)
    scaled1 = _scale(hw1, deg_p)
    p1 = _scatter_kernel(scaled1, src, dst, zerosd)
    scaled2 = _layer_mm(p1, deg_p, b1.reshape(1, D), W2)
    p2 = _scatter_kernel(scaled2, src, dst, zerosd)
    h = _layer_out(p2, deg_p, b2.reshape(1, D))
    uh = _user_gather_kernel(h, users)
    R = _mlp(uh, Ws1, bs1.reshape(1, -1), Ws2, bs2.reshape(1, -1))
    return (R, h)


# speedup vs baseline: 11.2634x; 1.3786x over previous
import functools

import jax
import jax.numpy as jnp
from jax import lax
from jax.experimental import pallas as pl
from jax.experimental.pallas import tpu as pltpu
from jax.experimental.pallas import tpu_sc as plsc

N = 10000
E = 320000
D = 128
NC = 2
NS = 16
NW = NC * NS

CHUNK = 128
NCHUNKS = E // CHUNK
BASE_CH = NCHUNKS // NW
EXTRA = NCHUNKS - BASE_CH * NW
STRIPE = 624
REM_BASE = NS * STRIPE
REM = N - REM_BASE
USERS_PER_W = 2048 // NW

_mesh = plsc.VectorSubcoreMesh(core_axis_name="c", subcore_axis_name="s")


def _worker_id():
    return lax.axis_index("s") * NC + lax.axis_index("c")


def _striped(s, fn):
    fn(s * STRIPE, STRIPE)

    @pl.when(s == NS - 1)
    def _():
        fn(REM_BASE, REM)


@functools.partial(
    pl.kernel,
    out_type=jax.ShapeDtypeStruct((NC, N, D), jnp.float32),
    mesh=_mesh,
    scratch_types=[
        pltpu.VMEM_SHARED((N, D), jnp.float32),
        pltpu.VMEM((CHUNK,), jnp.int32),
        pltpu.VMEM((CHUNK,), jnp.int32),
        pltpu.VMEM((CHUNK, D), jnp.float32),
        pltpu.VMEM((CHUNK, D), jnp.float32),
    ],
)
def _hist_kernel(src_hbm, dst_hbm, zerosd_hbm, ones_a_hbm, ones_b_hbm,
                 deg_out, acc, sidx, didx, ones_a, ones_b):
    c = lax.axis_index("c")
    s = lax.axis_index("s")
    w = _worker_id()

    def _zero(r0, nr):
        pltpu.sync_copy(zerosd_hbm.at[pl.ds(r0, nr)], acc.at[pl.ds(r0, nr)])

    _striped(s, _zero)
    pltpu.sync_copy(ones_a_hbm, ones_a)
    pltpu.sync_copy(ones_b_hbm, ones_b)
    plsc.subcore_barrier()

    n_my = BASE_CH + (w < EXTRA).astype(jnp.int32)

    @pl.loop(0, n_my)
    def _(i):
        e0 = (w + i * NW) * CHUNK
        pltpu.sync_copy(src_hbm.at[pl.ds(e0, CHUNK)], sidx)
        pltpu.sync_copy(dst_hbm.at[pl.ds(e0, CHUNK)], didx)
        pltpu.sync_copy(ones_a, acc.at[sidx], add=True)
        pltpu.sync_copy(ones_b, acc.at[didx], add=True)

    plsc.subcore_barrier()

    def _wout(r0, nr):
        pltpu.sync_copy(acc.at[pl.ds(r0, nr)], deg_out.at[c, pl.ds(r0, nr)])

    _striped(s, _wout)


NBUF = 3
NGROUPS = (BASE_CH + 1 + NBUF - 1) // NBUF


@functools.partial(
    pl.kernel,
    out_type=jax.ShapeDtypeStruct((NC, N, D), jnp.float32),
    mesh=_mesh,
    scratch_types=(
        [pltpu.VMEM_SHARED((N, D), jnp.float32)]
        + [pltpu.VMEM((CHUNK,), jnp.int32)] * NBUF
        + [pltpu.VMEM((CHUNK,), jnp.int32)] * NBUF
        + [pltpu.VMEM((CHUNK, D), jnp.float32)] * NBUF
        + [pltpu.SemaphoreType.DMA] * (4 * NBUF)
    ),
)
def _scatter_kernel(table_hbm, src_hbm, dst_hbm, zerosd_hbm,
                    out_hbm, acc, *bufs):
    sidx = bufs[0:NBUF]
    didx = bufs[NBUF:2 * NBUF]
    rows = bufs[2 * NBUF:3 * NBUF]
    sems = bufs[3 * NBUF:]
    sem_si = sems[0:NBUF]
    sem_di = sems[NBUF:2 * NBUF]
    sem_g = sems[2 * NBUF:3 * NBUF]
    sem_a = sems[3 * NBUF:4 * NBUF]

    c = lax.axis_index("c")
    s = lax.axis_index("s")
    w = _worker_id()

    def _zero(r0, nr):
        pltpu.sync_copy(zerosd_hbm.at[pl.ds(r0, nr)], acc.at[pl.ds(r0, nr)])

    _striped(s, _zero)
    plsc.subcore_barrier()

    n_my = BASE_CH + (w < EXTRA).astype(jnp.int32)

    def _start_idx(b, q):
        e0 = (w + q * NW) * CHUNK
        pltpu.async_copy(src_hbm.at[pl.ds(e0, CHUNK)], sidx[b], sem_si[b])
        pltpu.async_copy(dst_hbm.at[pl.ds(e0, CHUNK)], didx[b], sem_di[b])

    def _wait_idx(b):
        pltpu.make_async_copy(src_hbm.at[pl.ds(0, CHUNK)], sidx[b], sem_si[b]).wait()
        pltpu.make_async_copy(dst_hbm.at[pl.ds(0, CHUNK)], didx[b], sem_di[b]).wait()

    for b in range(NBUF):
        _start_idx(b, b)

    @pl.loop(0, NGROUPS)
    def _(g):
        q0 = g * NBUF
        for b in range(NBUF):
            @pl.when(q0 + b < n_my)
            def _(b=b):
                _wait_idx(b)
                pltpu.async_copy(table_hbm.at[sidx[b]], rows[b], sem_g[b])
        for b in range(NBUF):
            @pl.when(q0 + b < n_my)
            def _(b=b):
                pltpu.make_async_copy(table_hbm.at[sidx[b]], rows[b], sem_g[b]).wait()
                pltpu.async_copy(rows[b], acc.at[didx[b]], sem_a[b], add=True)
        for b in range(NBUF):
            @pl.when(q0 + b < n_my)
            def _(b=b):
                pltpu.make_async_copy(rows[b], acc.at[didx[b]], sem_a[b]).wait()

            @pl.when(q0 + b + NBUF < n_my)
            def _(b=b):
                _start_idx(b, q0 + b + NBUF)

    plsc.subcore_barrier()

    def _wout(r0, nr):
        pltpu.sync_copy(acc.at[pl.ds(r0, nr)], out_hbm.at[c, pl.ds(r0, nr)])

    _striped(s, _wout)


@functools.partial(
    pl.kernel,
    out_type=jax.ShapeDtypeStruct((2048, D), jnp.float32),
    mesh=_mesh,
    scratch_types=[
        pltpu.VMEM((USERS_PER_W,), jnp.int32),
        pltpu.VMEM((USERS_PER_W, D), jnp.float32),
        pltpu.SemaphoreType.DMA,
    ],
)
def _user_gather_kernel(h_hbm, users_hbm, out_hbm, uidx, rows, sem):
    w = _worker_id()
    base = w * USERS_PER_W
    pltpu.sync_copy(users_hbm.at[pl.ds(base, USERS_PER_W)], uidx)
    pltpu.async_copy(h_hbm.at[uidx], rows, sem).wait()
    pltpu.sync_copy(rows, out_hbm.at[pl.ds(base, USERS_PER_W)])


_BLK = 1000


def _mm_body(x_ref, w_ref, o_ref):
    o_ref[...] = jnp.dot(x_ref[...], w_ref[...],
                         preferred_element_type=jnp.float32)


def _mm(x, w):
    n = x.shape[0]
    return pl.pallas_call(
        _mm_body,
        grid=(n // _BLK,),
        in_specs=[
            pl.BlockSpec((_BLK, x.shape[1]), lambda i: (i, 0)),
            pl.BlockSpec(w.shape, lambda i: (0, 0)),
        ],
        out_specs=pl.BlockSpec((_BLK, w.shape[1]), lambda i: (i, 0)),
        out_shape=jax.ShapeDtypeStruct((n, w.shape[1]), jnp.float32),
    )(x, w)


def _norm_from(deg_ref, col):
    d = deg_ref[0, :, col:col + 1] + deg_ref[1, :, col:col + 1]
    return lax.rsqrt(jnp.maximum(d, 1.0))


def _scale_body(hw_ref, deg_ref, o_ref):
    o_ref[...] = hw_ref[...] * _norm_from(deg_ref, 0)


def _scale(hw, deg_p):
    return pl.pallas_call(
        _scale_body,
        grid=(N // _BLK,),
        in_specs=[
            pl.BlockSpec((_BLK, D), lambda i: (i, 0)),
            pl.BlockSpec((NC, _BLK, D), lambda i: (0, i, 0)),
        ],
        out_specs=pl.BlockSpec((_BLK, D), lambda i: (i, 0)),
        out_shape=jax.ShapeDtypeStruct((N, D), jnp.float32),
    )(hw, deg_p)


def _layer_mm_body(p_ref, deg_ref, b_ref, w_ref, o_ref):
    agg = p_ref[0] + p_ref[1]
    h = jax.nn.relu(agg * _norm_from(deg_ref, 1) + b_ref[...])
    o_ref[...] = jnp.dot(h, w_ref[...],
                         preferred_element_type=jnp.float32) * _norm_from(deg_ref, 0)


def _layer_mm(p, deg_p, b, w):
    return pl.pallas_call(
        _layer_mm_body,
        grid=(N // _BLK,),
        in_specs=[
            pl.BlockSpec((NC, _BLK, D), lambda i: (0, i, 0)),
            pl.BlockSpec((NC, _BLK, D), lambda i: (0, i, 0)),
            pl.BlockSpec((1, D), lambda i: (0, 0)),
            pl.BlockSpec((D, D), lambda i: (0, 0)),
        ],
        out_specs=pl.BlockSpec((_BLK, D), lambda i: (i, 0)),
        out_shape=jax.ShapeDtypeStruct((N, D), jnp.float32),
    )(p, deg_p, b, w)


def _layer_out_body(p_ref, deg_ref, b_ref, o_ref):
    agg = p_ref[0] + p_ref[1]
    o_ref[...] = jax.nn.relu(agg * _norm_from(deg_ref, 1) + b_ref[...])


def _layer_out(p, deg_p, b):
    return pl.pallas_call(
        _layer_out_body,
        grid=(N // _BLK,),
        in_specs=[
            pl.BlockSpec((NC, _BLK, D), lambda i: (0, i, 0)),
            pl.BlockSpec((NC, _BLK, D), lambda i: (0, i, 0)),
            pl.BlockSpec((1, D), lambda i: (0, 0)),
        ],
        out_specs=pl.BlockSpec((_BLK, D), lambda i: (i, 0)),
        out_shape=jax.ShapeDtypeStruct((N, D), jnp.float32),
    )(p, deg_p, b)


def _mlp_body(uh_ref, w1_ref, b1_ref, w2_ref, b2_ref, o_ref):
    t = jnp.tanh(jnp.dot(uh_ref[...], w1_ref[...],
                         preferred_element_type=jnp.float32) + b1_ref[...])
    o_ref[...] = jnp.dot(t, w2_ref[...],
                         preferred_element_type=jnp.float32) + b2_ref[...]


def _mlp(uh, w1, b1, w2, b2):
    return pl.pallas_call(
        _mlp_body,
        out_shape=jax.ShapeDtypeStruct((uh.shape[0], w2.shape[1]), jnp.float32),
    )(uh, w1, b1, w2, b2)


def kernel(features, W1, b1, W2, b2, Ws1, bs1, Ws2, bs2, edge_index, users):
    src = edge_index[0].astype(jnp.int32)
    dst = edge_index[1].astype(jnp.int32)
    users = users.astype(jnp.int32)
    zerosd = jnp.zeros((N, D), jnp.float32)
    col = jnp.arange(D, dtype=jnp.int32)[None, :]
    ones_a = jnp.broadcast_to((col == 0).astype(jnp.float32), (CHUNK, D))
    ones_b = jnp.broadcast_to((col == 1).astype(jnp.float32), (CHUNK, D))

    deg_p = _hist_kernel(src, dst, zerosd, ones_a, ones_b)
    hw1 = _mm(features, W1)
    scaled1 = _scale(hw1, deg_p)
    p1 = _scatter_kernel(scaled1, src, dst, zerosd)
    scaled2 = _layer_mm(p1, deg_p, b1.reshape(1, D), W2)
    p2 = _scatter_kernel(scaled2, src, dst, zerosd)
    h = _layer_out(p2, deg_p, b2.reshape(1, D))
    uh = _user_gather_kernel(h, users)
    R = _mlp(uh, Ws1, bs1.reshape(1, -1), Ws2, bs2.reshape(1, -1))
    return (R, h)


# speedup vs baseline: 12.8767x; 1.1432x over previous
import functools

import jax
import jax.numpy as jnp
from jax import lax
from jax.experimental import pallas as pl
from jax.experimental.pallas import tpu as pltpu
from jax.experimental.pallas import tpu_sc as plsc

N = 10000
E = 320000
D = 128
NC = 2
NS = 16
NW = NC * NS

CHUNK = 128
NCHUNKS = E // CHUNK
BASE_CH = NCHUNKS // NW
EXTRA = NCHUNKS - BASE_CH * NW
STRIPE = 624
REM_BASE = NS * STRIPE
REM = N - REM_BASE
USERS_PER_W = 2048 // NW

_mesh = plsc.VectorSubcoreMesh(core_axis_name="c", subcore_axis_name="s")


def _worker_id():
    return lax.axis_index("s") * NC + lax.axis_index("c")


def _striped(s, fn):
    fn(s * STRIPE, STRIPE)

    @pl.when(s == NS - 1)
    def _():
        fn(REM_BASE, REM)


HBUF = 4
HGROUPS = (2500 // NW + 1 + HBUF - 1) // HBUF


@functools.partial(
    pl.kernel,
    out_type=jax.ShapeDtypeStruct((NC, N, D), jnp.float32),
    mesh=_mesh,
    scratch_types=(
        [pltpu.VMEM_SHARED((N, D), jnp.float32)]
        + [pltpu.VMEM((CHUNK, D), jnp.float32)] * 2
        + [pltpu.VMEM((CHUNK,), jnp.int32)] * (2 * HBUF)
        + [pltpu.SemaphoreType.DMA] * (4 * HBUF)
    ),
)
def _hist_kernel(src_hbm, dst_hbm, zerosd_hbm, ones_a_hbm, ones_b_hbm,
                 deg_out, acc, ones_a, ones_b, *bufs):
    sidx = bufs[0:HBUF]
    didx = bufs[HBUF:2 * HBUF]
    sems = bufs[2 * HBUF:]
    sem_si = sems[0:HBUF]
    sem_di = sems[HBUF:2 * HBUF]
    sem_as = sems[2 * HBUF:3 * HBUF]
    sem_ad = sems[3 * HBUF:4 * HBUF]

    c = lax.axis_index("c")
    s = lax.axis_index("s")
    w = _worker_id()

    def _zero(r0, nr):
        pltpu.sync_copy(zerosd_hbm.at[pl.ds(r0, nr)], acc.at[pl.ds(r0, nr)])

    _striped(s, _zero)
    pltpu.sync_copy(ones_a_hbm, ones_a)
    pltpu.sync_copy(ones_b_hbm, ones_b)
    plsc.subcore_barrier()

    n_my = BASE_CH + (w < EXTRA).astype(jnp.int32)

    def _start_idx(b, q):
        e0 = (w + q * NW) * CHUNK
        pltpu.async_copy(src_hbm.at[pl.ds(e0, CHUNK)], sidx[b], sem_si[b])
        pltpu.async_copy(dst_hbm.at[pl.ds(e0, CHUNK)], didx[b], sem_di[b])

    for b in range(HBUF):
        _start_idx(b, b)

    @pl.loop(0, HGROUPS)
    def _(g):
        q0 = g * HBUF
        for b in range(HBUF):
            @pl.when(q0 + b < n_my)
            def _(b=b):
                pltpu.make_async_copy(src_hbm.at[pl.ds(0, CHUNK)], sidx[b],
                                      sem_si[b]).wait()
                pltpu.make_async_copy(dst_hbm.at[pl.ds(0, CHUNK)], didx[b],
                                      sem_di[b]).wait()
                pltpu.async_copy(ones_a, acc.at[sidx[b]], sem_as[b], add=True)
                pltpu.async_copy(ones_b, acc.at[didx[b]], sem_ad[b], add=True)
        for b in range(HBUF):
            @pl.when(q0 + b < n_my)
            def _(b=b):
                pltpu.make_async_copy(ones_a, acc.at[sidx[b]], sem_as[b]).wait()
                pltpu.make_async_copy(ones_b, acc.at[didx[b]], sem_ad[b]).wait()

            @pl.when(q0 + b + HBUF < n_my)
            def _(b=b):
                _start_idx(b, q0 + b + HBUF)

    plsc.subcore_barrier()

    def _wout(r0, nr):
        pltpu.sync_copy(acc.at[pl.ds(r0, nr)], deg_out.at[c, pl.ds(r0, nr)])

    _striped(s, _wout)


NBUF = 3
NGROUPS = (BASE_CH + 1 + NBUF - 1) // NBUF


@functools.partial(
    pl.kernel,
    out_type=jax.ShapeDtypeStruct((NC, N, D), jnp.float32),
    mesh=_mesh,
    scratch_types=(
        [pltpu.VMEM_SHARED((N, D), jnp.float32)]
        + [pltpu.VMEM((CHUNK,), jnp.int32)] * NBUF
        + [pltpu.VMEM((CHUNK,), jnp.int32)] * NBUF
        + [pltpu.VMEM((CHUNK, D), jnp.float32)] * NBUF
        + [pltpu.SemaphoreType.DMA] * (4 * NBUF)
    ),
)
def _scatter_kernel(table_hbm, src_hbm, dst_hbm, zerosd_hbm,
                    out_hbm, acc, *bufs):
    sidx = bufs[0:NBUF]
    didx = bufs[NBUF:2 * NBUF]
    rows = bufs[2 * NBUF:3 * NBUF]
    sems = bufs[3 * NBUF:]
    sem_si = sems[0:NBUF]
    sem_di = sems[NBUF:2 * NBUF]
    sem_g = sems[2 * NBUF:3 * NBUF]
    sem_a = sems[3 * NBUF:4 * NBUF]

    c = lax.axis_index("c")
    s = lax.axis_index("s")
    w = _worker_id()

    def _zero(r0, nr):
        pltpu.sync_copy(zerosd_hbm.at[pl.ds(r0, nr)], acc.at[pl.ds(r0, nr)])

    _striped(s, _zero)
    plsc.subcore_barrier()

    n_my = BASE_CH + (w < EXTRA).astype(jnp.int32)

    def _start_idx(b, q):
        e0 = (w + q * NW) * CHUNK
        pltpu.async_copy(src_hbm.at[pl.ds(e0, CHUNK)], sidx[b], sem_si[b])
        pltpu.async_copy(dst_hbm.at[pl.ds(e0, CHUNK)], didx[b], sem_di[b])

    def _wait_idx(b):
        pltpu.make_async_copy(src_hbm.at[pl.ds(0, CHUNK)], sidx[b], sem_si[b]).wait()
        pltpu.make_async_copy(dst_hbm.at[pl.ds(0, CHUNK)], didx[b], sem_di[b]).wait()

    for b in range(NBUF):
        _start_idx(b, b)

    @pl.loop(0, NGROUPS)
    def _(g):
        q0 = g * NBUF
        for b in range(NBUF):
            @pl.when(q0 + b < n_my)
            def _(b=b):
                _wait_idx(b)
                pltpu.async_copy(table_hbm.at[sidx[b]], rows[b], sem_g[b])
        for b in range(NBUF):
            @pl.when(q0 + b < n_my)
            def _(b=b):
                pltpu.make_async_copy(table_hbm.at[sidx[b]], rows[b], sem_g[b]).wait()
                pltpu.async_copy(rows[b], acc.at[didx[b]], sem_a[b], add=True)
        for b in range(NBUF):
            @pl.when(q0 + b < n_my)
            def _(b=b):
                pltpu.make_async_copy(rows[b], acc.at[didx[b]], sem_a[b]).wait()

            @pl.when(q0 + b + NBUF < n_my)
            def _(b=b):
                _start_idx(b, q0 + b + NBUF)

    plsc.subcore_barrier()

    def _wout(r0, nr):
        pltpu.sync_copy(acc.at[pl.ds(r0, nr)], out_hbm.at[c, pl.ds(r0, nr)])

    _striped(s, _wout)


@functools.partial(
    pl.kernel,
    out_type=jax.ShapeDtypeStruct((2048, D), jnp.float32),
    mesh=_mesh,
    scratch_types=[
        pltpu.VMEM((USERS_PER_W,), jnp.int32),
        pltpu.VMEM((USERS_PER_W, D), jnp.float32),
        pltpu.SemaphoreType.DMA,
    ],
)
def _user_gather_kernel(h_hbm, users_hbm, out_hbm, uidx, rows, sem):
    w = _worker_id()
    base = w * USERS_PER_W
    pltpu.sync_copy(users_hbm.at[pl.ds(base, USERS_PER_W)], uidx)
    pltpu.async_copy(h_hbm.at[uidx], rows, sem).wait()
    pltpu.sync_copy(rows, out_hbm.at[pl.ds(base, USERS_PER_W)])


_BLK = 1000


def _mm_body(x_ref, w_ref, o_ref):
    o_ref[...] = jnp.dot(x_ref[...], w_ref[...],
                         preferred_element_type=jnp.float32)


def _mm(x, w):
    n = x.shape[0]
    return pl.pallas_call(
        _mm_body,
        grid=(n // _BLK,),
        in_specs=[
            pl.BlockSpec((_BLK, x.shape[1]), lambda i: (i, 0)),
            pl.BlockSpec(w.shape, lambda i: (0, 0)),
        ],
        out_specs=pl.BlockSpec((_BLK, w.shape[1]), lambda i: (i, 0)),
        out_shape=jax.ShapeDtypeStruct((n, w.shape[1]), jnp.float32),
    )(x, w)


def _norm_from(deg_ref, col):
    d = deg_ref[0, :, col:col + 1] + deg_ref[1, :, col:col + 1]
    return lax.rsqrt(jnp.maximum(d, 1.0))


def _scale_body(hw_ref, deg_ref, o_ref):
    o_ref[...] = hw_ref[...] * _norm_from(deg_ref, 0)


def _scale(hw, deg_p):
    return pl.pallas_call(
        _scale_body,
        grid=(N // _BLK,),
        in_specs=[
            pl.BlockSpec((_BLK, D), lambda i: (i, 0)),
            pl.BlockSpec((NC, _BLK, D), lambda i: (0, i, 0)),
        ],
        out_specs=pl.BlockSpec((_BLK, D), lambda i: (i, 0)),
        out_shape=jax.ShapeDtypeStruct((N, D), jnp.float32),
    )(hw, deg_p)


def _layer_mm_body(p_ref, deg_ref, b_ref, w_ref, o_ref):
    agg = p_ref[0] + p_ref[1]
    h = jax.nn.relu(agg * _norm_from(deg_ref, 1) + b_ref[...])
    o_ref[...] = jnp.dot(h, w_ref[...],
                         preferred_element_type=jnp.float32) * _norm_from(deg_ref, 0)


def _layer_mm(p, deg_p, b, w):
    return pl.pallas_call(
        _layer_mm_body,
        grid=(N // _BLK,),
        in_specs=[
            pl.BlockSpec((NC, _BLK, D), lambda i: (0, i, 0)),
            pl.BlockSpec((NC, _BLK, D), lambda i: (0, i, 0)),
            pl.BlockSpec((1, D), lambda i: (0, 0)),
            pl.BlockSpec((D, D), lambda i: (0, 0)),
        ],
        out_specs=pl.BlockSpec((_BLK, D), lambda i: (i, 0)),
        out_shape=jax.ShapeDtypeStruct((N, D), jnp.float32),
    )(p, deg_p, b, w)


def _layer_out_body(p_ref, deg_ref, b_ref, o_ref):
    agg = p_ref[0] + p_ref[1]
    o_ref[...] = jax.nn.relu(agg * _norm_from(deg_ref, 1) + b_ref[...])


def _layer_out(p, deg_p, b):
    return pl.pallas_call(
        _layer_out_body,
        grid=(N // _BLK,),
        in_specs=[
            pl.BlockSpec((NC, _BLK, D), lambda i: (0, i, 0)),
            pl.BlockSpec((NC, _BLK, D), lambda i: (0, i, 0)),
            pl.BlockSpec((1, D), lambda i: (0, 0)),
        ],
        out_specs=pl.BlockSpec((_BLK, D), lambda i: (i, 0)),
        out_shape=jax.ShapeDtypeStruct((N, D), jnp.float32),
    )(p, deg_p, b)


def _mlp_body(uh_ref, w1_ref, b1_ref, w2_ref, b2_ref, o_ref):
    t = jnp.tanh(jnp.dot(uh_ref[...], w1_ref[...],
                         preferred_element_type=jnp.float32) + b1_ref[...])
    o_ref[...] = jnp.dot(t, w2_ref[...],
                         preferred_element_type=jnp.float32) + b2_ref[...]


def _mlp(uh, w1, b1, w2, b2):
    return pl.pallas_call(
        _mlp_body,
        out_shape=jax.ShapeDtypeStruct((uh.shape[0], w2.shape[1]), jnp.float32),
    )(uh, w1, b1, w2, b2)


def kernel(features, W1, b1, W2, b2, Ws1, bs1, Ws2, bs2, edge_index, users):
    src = edge_index[0].astype(jnp.int32)
    dst = edge_index[1].astype(jnp.int32)
    users = users.astype(jnp.int32)
    zerosd = jnp.zeros((N, D), jnp.float32)
    col = jnp.arange(D, dtype=jnp.int32)[None, :]
    ones_a = jnp.broadcast_to((col == 0).astype(jnp.float32), (CHUNK, D))
    ones_b = jnp.broadcast_to((col == 1).astype(jnp.float32), (CHUNK, D))

    deg_p = _hist_kernel(src, dst, zerosd, ones_a, ones_b)
    hw1 = _mm(features, W1)
    scaled1 = _scale(hw1, deg_p)
    p1 = _scatter_kernel(scaled1, src, dst, zerosd)
    scaled2 = _layer_mm(p1, deg_p, b1.reshape(1, D), W2)
    p2 = _scatter_kernel(scaled2, src, dst, zerosd)
    h = _layer_out(p2, deg_p, b2.reshape(1, D))
    uh = _user_gather_kernel(h, users)
    R = _mlp(uh, Ws1, bs1.reshape(1, -1), Ws2, bs2.reshape(1, -1))
    return (R, h)


# speedup vs baseline: 15.8689x; 1.2324x over previous
import functools

import jax
import jax.numpy as jnp
from jax import lax
from jax.experimental import pallas as pl
from jax.experimental.pallas import tpu as pltpu
from jax.experimental.pallas import tpu_sc as plsc

N = 10000
E = 320000
D = 128
NC = 2
NS = 16
NW = NC * NS

CHUNK = 128
NCHUNKS = E // CHUNK
BASE_CH = NCHUNKS // NW
EXTRA = NCHUNKS - BASE_CH * NW
STRIPE = 624
REM_BASE = NS * STRIPE
REM = N - REM_BASE
USERS_PER_W = 2048 // NW

_mesh = plsc.VectorSubcoreMesh(core_axis_name="c", subcore_axis_name="s")


def _worker_id():
    return lax.axis_index("s") * NC + lax.axis_index("c")


def _striped(s, fn):
    fn(s * STRIPE, STRIPE)

    @pl.when(s == NS - 1)
    def _():
        fn(REM_BASE, REM)


HBUF = 6
HGROUPS = (2500 // NW + 1 + HBUF - 1) // HBUF


@functools.partial(
    pl.kernel,
    out_type=(
        jax.ShapeDtypeStruct((NC, N, 16), jnp.float32),
        jax.ShapeDtypeStruct((NC, N, 16), jnp.float32),
    ),
    mesh=_mesh,
    compiler_params=pltpu.CompilerParams(use_tc_tiling_on_sc=False),
    scratch_types=(
        [pltpu.VMEM_SHARED((N, 16), jnp.float32)] * 2
        + [pltpu.VMEM((CHUNK, 16), jnp.float32)]
        + [pltpu.VMEM((CHUNK,), jnp.int32)] * (2 * HBUF)
        + [pltpu.SemaphoreType.DMA] * (4 * HBUF)
    ),
)
def _hist_kernel(src_hbm, dst_hbm, zeros16_hbm, ones_hbm,
                 degs_out, degd_out, acc_s, acc_d, ones_v, *bufs):
    sidx = bufs[0:HBUF]
    didx = bufs[HBUF:2 * HBUF]
    sems = bufs[2 * HBUF:]
    sem_si = sems[0:HBUF]
    sem_di = sems[HBUF:2 * HBUF]
    sem_as = sems[2 * HBUF:3 * HBUF]
    sem_ad = sems[3 * HBUF:4 * HBUF]

    c = lax.axis_index("c")
    s = lax.axis_index("s")
    w = _worker_id()

    def _zero(r0, nr):
        pltpu.sync_copy(zeros16_hbm.at[pl.ds(r0, nr)], acc_s.at[pl.ds(r0, nr)])
        pltpu.sync_copy(zeros16_hbm.at[pl.ds(r0, nr)], acc_d.at[pl.ds(r0, nr)])

    _striped(s, _zero)
    pltpu.sync_copy(ones_hbm, ones_v)
    plsc.subcore_barrier()

    n_my = BASE_CH + (w < EXTRA).astype(jnp.int32)

    def _start_idx(b, q):
        e0 = (w + q * NW) * CHUNK
        pltpu.async_copy(src_hbm.at[pl.ds(e0, CHUNK)], sidx[b], sem_si[b])
        pltpu.async_copy(dst_hbm.at[pl.ds(e0, CHUNK)], didx[b], sem_di[b])

    for b in range(HBUF):
        _start_idx(b, b)

    @pl.loop(0, HGROUPS)
    def _(g):
        q0 = g * HBUF
        for b in range(HBUF):
            @pl.when(q0 + b < n_my)
            def _(b=b):
                pltpu.make_async_copy(src_hbm.at[pl.ds(0, CHUNK)], sidx[b],
                                      sem_si[b]).wait()
                pltpu.make_async_copy(dst_hbm.at[pl.ds(0, CHUNK)], didx[b],
                                      sem_di[b]).wait()
                pltpu.async_copy(ones_v, acc_s.at[sidx[b]], sem_as[b], add=True)
                pltpu.async_copy(ones_v, acc_d.at[didx[b]], sem_ad[b], add=True)
        for b in range(HBUF):
            @pl.when(q0 + b < n_my)
            def _(b=b):
                pltpu.make_async_copy(ones_v, acc_s.at[sidx[b]], sem_as[b]).wait()
                pltpu.make_async_copy(ones_v, acc_d.at[didx[b]], sem_ad[b]).wait()

            @pl.when(q0 + b + HBUF < n_my)
            def _(b=b):
                _start_idx(b, q0 + b + HBUF)

    plsc.subcore_barrier()

    def _wout(r0, nr):
        pltpu.sync_copy(acc_s.at[pl.ds(r0, nr)], degs_out.at[c, pl.ds(r0, nr)])
        pltpu.sync_copy(acc_d.at[pl.ds(r0, nr)], degd_out.at[c, pl.ds(r0, nr)])

    _striped(s, _wout)


NBUF = 3
NGROUPS = (BASE_CH + 1 + NBUF - 1) // NBUF


@functools.partial(
    pl.kernel,
    out_type=jax.ShapeDtypeStruct((NC, N, D), jnp.float32),
    mesh=_mesh,
    scratch_types=(
        [pltpu.VMEM_SHARED((N, D), jnp.float32)]
        + [pltpu.VMEM((CHUNK,), jnp.int32)] * NBUF
        + [pltpu.VMEM((CHUNK,), jnp.int32)] * NBUF
        + [pltpu.VMEM((CHUNK, D), jnp.float32)] * NBUF
        + [pltpu.SemaphoreType.DMA] * (4 * NBUF)
    ),
)
def _scatter_kernel(table_hbm, src_hbm, dst_hbm, zerosd_hbm,
                    out_hbm, acc, *bufs):
    sidx = bufs[0:NBUF]
    didx = bufs[NBUF:2 * NBUF]
    rows = bufs[2 * NBUF:3 * NBUF]
    sems = bufs[3 * NBUF:]
    sem_si = sems[0:NBUF]
    sem_di = sems[NBUF:2 * NBUF]
    sem_g = sems[2 * NBUF:3 * NBUF]
    sem_a = sems[3 * NBUF:4 * NBUF]

    c = lax.axis_index("c")
    s = lax.axis_index("s")
    w = _worker_id()

    def _zero(r0, nr):
        pltpu.sync_copy(zerosd_hbm.at[pl.ds(r0, nr)], acc.at[pl.ds(r0, nr)])

    _striped(s, _zero)
    plsc.subcore_barrier()

    n_my = BASE_CH + (w < EXTRA).astype(jnp.int32)

    def _start_idx(b, q):
        e0 = (w + q * NW) * CHUNK
        pltpu.async_copy(src_hbm.at[pl.ds(e0, CHUNK)], sidx[b], sem_si[b])
        pltpu.async_copy(dst_hbm.at[pl.ds(e0, CHUNK)], didx[b], sem_di[b])

    def _wait_idx(b):
        pltpu.make_async_copy(src_hbm.at[pl.ds(0, CHUNK)], sidx[b], sem_si[b]).wait()
        pltpu.make_async_copy(dst_hbm.at[pl.ds(0, CHUNK)], didx[b], sem_di[b]).wait()

    for b in range(NBUF):
        _start_idx(b, b)

    @pl.loop(0, NGROUPS)
    def _(g):
        q0 = g * NBUF
        for b in range(NBUF):
            @pl.when(q0 + b < n_my)
            def _(b=b):
                _wait_idx(b)
                pltpu.async_copy(table_hbm.at[sidx[b]], rows[b], sem_g[b])
        for b in range(NBUF):
            @pl.when(q0 + b < n_my)
            def _(b=b):
                pltpu.make_async_copy(table_hbm.at[sidx[b]], rows[b], sem_g[b]).wait()
                pltpu.async_copy(rows[b], acc.at[didx[b]], sem_a[b], add=True)
        for b in range(NBUF):
            @pl.when(q0 + b < n_my)
            def _(b=b):
                pltpu.make_async_copy(rows[b], acc.at[didx[b]], sem_a[b]).wait()

            @pl.when(q0 + b + NBUF < n_my)
            def _(b=b):
                _start_idx(b, q0 + b + NBUF)

    plsc.subcore_barrier()

    def _wout(r0, nr):
        pltpu.sync_copy(acc.at[pl.ds(r0, nr)], out_hbm.at[c, pl.ds(r0, nr)])

    _striped(s, _wout)


@functools.partial(
    pl.kernel,
    out_type=jax.ShapeDtypeStruct((2048, D), jnp.float32),
    mesh=_mesh,
    scratch_types=[
        pltpu.VMEM((USERS_PER_W,), jnp.int32),
        pltpu.VMEM((USERS_PER_W, D), jnp.float32),
        pltpu.SemaphoreType.DMA,
    ],
)
def _user_gather_kernel(h_hbm, users_hbm, out_hbm, uidx, rows, sem):
    w = _worker_id()
    base = w * USERS_PER_W
    pltpu.sync_copy(users_hbm.at[pl.ds(base, USERS_PER_W)], uidx)
    pltpu.async_copy(h_hbm.at[uidx], rows, sem).wait()
    pltpu.sync_copy(rows, out_hbm.at[pl.ds(base, USERS_PER_W)])


_BLK = 1000


def _mm_body(x_ref, w_ref, o_ref):
    o_ref[...] = jnp.dot(x_ref[...], w_ref[...],
                         preferred_element_type=jnp.float32)


def _mm(x, w):
    n = x.shape[0]
    return pl.pallas_call(
        _mm_body,
        grid=(n // _BLK,),
        in_specs=[
            pl.BlockSpec((_BLK, x.shape[1]), lambda i: (i, 0)),
            pl.BlockSpec(w.shape, lambda i: (0, 0)),
        ],
        out_specs=pl.BlockSpec((_BLK, w.shape[1]), lambda i: (i, 0)),
        out_shape=jax.ShapeDtypeStruct((n, w.shape[1]), jnp.float32),
    )(x, w)


def _norm_from(deg_ref):
    d = deg_ref[0, :, 0:1] + deg_ref[1, :, 0:1]
    return lax.rsqrt(jnp.maximum(d, 1.0))


_DEG_SPEC = pl.BlockSpec((NC, _BLK, 16), lambda i: (0, i, 0))


def _scale_body(hw_ref, degs_ref, o_ref):
    o_ref[...] = hw_ref[...] * _norm_from(degs_ref)


def _scale(hw, degs_p):
    return pl.pallas_call(
        _scale_body,
        grid=(N // _BLK,),
        in_specs=[
            pl.BlockSpec((_BLK, D), lambda i: (i, 0)),
            _DEG_SPEC,
        ],
        out_specs=pl.BlockSpec((_BLK, D), lambda i: (i, 0)),
        out_shape=jax.ShapeDtypeStruct((N, D), jnp.float32),
    )(hw, degs_p)


def _layer_mm_body(p_ref, degd_ref, degs_ref, b_ref, w_ref, o_ref):
    agg = p_ref[0] + p_ref[1]
    h = jax.nn.relu(agg * _norm_from(degd_ref) + b_ref[...])
    o_ref[...] = jnp.dot(h, w_ref[...],
                         preferred_element_type=jnp.float32) * _norm_from(degs_ref)


def _layer_mm(p, degd_p, degs_p, b, w):
    return pl.pallas_call(
        _layer_mm_body,
        grid=(N // _BLK,),
        in_specs=[
            pl.BlockSpec((NC, _BLK, D), lambda i: (0, i, 0)),
            _DEG_SPEC,
            _DEG_SPEC,
            pl.BlockSpec((1, D), lambda i: (0, 0)),
            pl.BlockSpec((D, D), lambda i: (0, 0)),
        ],
        out_specs=pl.BlockSpec((_BLK, D), lambda i: (i, 0)),
        out_shape=jax.ShapeDtypeStruct((N, D), jnp.float32),
    )(p, degd_p, degs_p, b, w)


def _layer_out_body(p_ref, degd_ref, b_ref, o_ref):
    agg = p_ref[0] + p_ref[1]
    o_ref[...] = jax.nn.relu(agg * _norm_from(degd_ref) + b_ref[...])


def _layer_out(p, degd_p, b):
    return pl.pallas_call(
        _layer_out_body,
        grid=(N // _BLK,),
        in_specs=[
            pl.BlockSpec((NC, _BLK, D), lambda i: (0, i, 0)),
            _DEG_SPEC,
            pl.BlockSpec((1, D), lambda i: (0, 0)),
        ],
        out_specs=pl.BlockSpec((_BLK, D), lambda i: (i, 0)),
        out_shape=jax.ShapeDtypeStruct((N, D), jnp.float32),
    )(p, degd_p, b)


def _mlp_body(uh_ref, w1_ref, b1_ref, w2_ref, b2_ref, o_ref):
    t = jnp.tanh(jnp.dot(uh_ref[...], w1_ref[...],
                         preferred_element_type=jnp.float32) + b1_ref[...])
    o_ref[...] = jnp.dot(t, w2_ref[...],
                         preferred_element_type=jnp.float32) + b2_ref[...]


def _mlp(uh, w1, b1, w2, b2):
    return pl.pallas_call(
        _mlp_body,
        out_shape=jax.ShapeDtypeStruct((uh.shape[0], w2.shape[1]), jnp.float32),
    )(uh, w1, b1, w2, b2)


def kernel(features, W1, b1, W2, b2, Ws1, bs1, Ws2, bs2, edge_index, users):
    src = edge_index[0].astype(jnp.int32)
    dst = edge_index[1].astype(jnp.int32)
    users = users.astype(jnp.int32)
    zerosd = jnp.zeros((N, D), jnp.float32)
    zeros16 = jnp.zeros((N, 16), jnp.float32)
    ones16 = jnp.ones((CHUNK, 16), jnp.float32)

    degs_p, degd_p = _hist_kernel(src, dst, zeros16, ones16)
    hw1 = _mm(features, W1)
    scaled1 = _scale(hw1, degs_p)
    p1 = _scatter_kernel(scaled1, src, dst, zerosd)
    scaled2 = _layer_mm(p1, degd_p, degs_p, b1.reshape(1, D), W2)
    p2 = _scatter_kernel(scaled2, src, dst, zerosd)
    h = _layer_out(p2, degd_p, b2.reshape(1, D))
    uh = _user_gather_kernel(h, users)
    R = _mlp(uh, Ws1, bs1.reshape(1, -1), Ws2, bs2.reshape(1, -1))
    return (R, h)


# speedup vs baseline: 17.7046x; 1.1157x over previous
import functools

import jax
import jax.numpy as jnp
from jax import lax
from jax.experimental import pallas as pl
from jax.experimental.pallas import tpu as pltpu
from jax.experimental.pallas import tpu_sc as plsc

N = 10000
E = 320000
D = 128
NC = 2
NS = 16
NW = NC * NS

CHUNK = 128
NCHUNKS = E // CHUNK
BASE_CH = NCHUNKS // NW
EXTRA = NCHUNKS - BASE_CH * NW
STRIPE = 624
REM_BASE = NS * STRIPE
REM = N - REM_BASE
USERS_PER_W = 2048 // NW

_mesh = plsc.VectorSubcoreMesh(core_axis_name="c", subcore_axis_name="s")


def _worker_id():
    return lax.axis_index("s") * NC + lax.axis_index("c")


def _striped(s, fn):
    fn(s * STRIPE, STRIPE)

    @pl.when(s == NS - 1)
    def _():
        fn(REM_BASE, REM)


HBUF = 6
HGROUPS = (2500 // NW + 1 + HBUF - 1) // HBUF


@functools.partial(
    pl.kernel,
    out_type=(
        jax.ShapeDtypeStruct((NC, N, 16), jnp.float32),
        jax.ShapeDtypeStruct((NC, N, 16), jnp.float32),
    ),
    mesh=_mesh,
    compiler_params=pltpu.CompilerParams(use_tc_tiling_on_sc=False),
    scratch_types=(
        [pltpu.VMEM_SHARED((N, 16), jnp.float32)] * 2
        + [pltpu.VMEM((CHUNK, 16), jnp.float32)]
        + [pltpu.VMEM((CHUNK,), jnp.int32)] * (2 * HBUF)
        + [pltpu.SemaphoreType.DMA] * (4 * HBUF)
    ),
)
def _hist_kernel(src_hbm, dst_hbm, zeros16_hbm, ones_hbm,
                 degs_out, degd_out, acc_s, acc_d, ones_v, *bufs):
    sidx = bufs[0:HBUF]
    didx = bufs[HBUF:2 * HBUF]
    sems = bufs[2 * HBUF:]
    sem_si = sems[0:HBUF]
    sem_di = sems[HBUF:2 * HBUF]
    sem_as = sems[2 * HBUF:3 * HBUF]
    sem_ad = sems[3 * HBUF:4 * HBUF]

    c = lax.axis_index("c")
    s = lax.axis_index("s")
    w = _worker_id()

    def _zero(r0, nr):
        pltpu.sync_copy(zeros16_hbm.at[pl.ds(r0, nr)], acc_s.at[pl.ds(r0, nr)])
        pltpu.sync_copy(zeros16_hbm.at[pl.ds(r0, nr)], acc_d.at[pl.ds(r0, nr)])

    _striped(s, _zero)
    pltpu.sync_copy(ones_hbm, ones_v)
    plsc.subcore_barrier()

    n_my = BASE_CH + (w < EXTRA).astype(jnp.int32)

    def _start_idx(b, q):
        e0 = (w + q * NW) * CHUNK
        pltpu.async_copy(src_hbm.at[pl.ds(e0, CHUNK)], sidx[b], sem_si[b])
        pltpu.async_copy(dst_hbm.at[pl.ds(e0, CHUNK)], didx[b], sem_di[b])

    for b in range(HBUF):
        _start_idx(b, b)

    @pl.loop(0, HGROUPS)
    def _(g):
        q0 = g * HBUF
        for b in range(HBUF):
            @pl.when(q0 + b < n_my)
            def _(b=b):
                pltpu.make_async_copy(src_hbm.at[pl.ds(0, CHUNK)], sidx[b],
                                      sem_si[b]).wait()
                pltpu.make_async_copy(dst_hbm.at[pl.ds(0, CHUNK)], didx[b],
                                      sem_di[b]).wait()
                pltpu.async_copy(ones_v, acc_s.at[sidx[b]], sem_as[b], add=True)
                pltpu.async_copy(ones_v, acc_d.at[didx[b]], sem_ad[b], add=True)
        for b in range(HBUF):
            @pl.when(q0 + b < n_my)
            def _(b=b):
                pltpu.make_async_copy(ones_v, acc_s.at[sidx[b]], sem_as[b]).wait()
                pltpu.make_async_copy(ones_v, acc_d.at[didx[b]], sem_ad[b]).wait()

            @pl.when(q0 + b + HBUF < n_my)
            def _(b=b):
                _start_idx(b, q0 + b + HBUF)

    plsc.subcore_barrier()

    def _wout(r0, nr):
        pltpu.sync_copy(acc_s.at[pl.ds(r0, nr)], degs_out.at[c, pl.ds(r0, nr)])
        pltpu.sync_copy(acc_d.at[pl.ds(r0, nr)], degd_out.at[c, pl.ds(r0, nr)])

    _striped(s, _wout)


NBUF = 3
NGROUPS = (BASE_CH + 1 + NBUF - 1) // NBUF + 1
NGROUPS += NGROUPS % 2


@functools.partial(
    pl.kernel,
    out_type=jax.ShapeDtypeStruct((NC, N, D), jnp.float32),
    mesh=_mesh,
    scratch_types=(
        [pltpu.VMEM_SHARED((N, D), jnp.float32)]
        + [pltpu.VMEM((CHUNK,), jnp.int32)] * (2 * NBUF)
        + [pltpu.VMEM((CHUNK,), jnp.int32)] * (2 * NBUF)
        + [pltpu.VMEM((CHUNK, D), jnp.float32)] * NBUF
        + [pltpu.SemaphoreType.DMA] * (6 * NBUF)
    ),
)
def _scatter_kernel(table_hbm, src_hbm, dst_hbm, zerosd_hbm,
                    out_hbm, acc, *bufs):
    sidx = (bufs[0:NBUF], bufs[NBUF:2 * NBUF])
    didx = (bufs[2 * NBUF:3 * NBUF], bufs[3 * NBUF:4 * NBUF])
    rows = bufs[4 * NBUF:5 * NBUF]
    sems = bufs[5 * NBUF:]
    sem_si = (sems[0:NBUF], sems[NBUF:2 * NBUF])
    sem_di = (sems[2 * NBUF:3 * NBUF], sems[3 * NBUF:4 * NBUF])
    sem_g = sems[4 * NBUF:5 * NBUF]
    sem_a = sems[5 * NBUF:6 * NBUF]

    c = lax.axis_index("c")
    s = lax.axis_index("s")
    w = _worker_id()

    def _zero(r0, nr):
        pltpu.sync_copy(zerosd_hbm.at[pl.ds(r0, nr)], acc.at[pl.ds(r0, nr)])

    _striped(s, _zero)
    plsc.subcore_barrier()

    n_my = BASE_CH + (w < EXTRA).astype(jnp.int32)

    def _start_idx(st, b, q):
        e0 = (w + q * NW) * CHUNK
        pltpu.async_copy(src_hbm.at[pl.ds(e0, CHUNK)], sidx[st][b], sem_si[st][b])
        pltpu.async_copy(dst_hbm.at[pl.ds(e0, CHUNK)], didx[st][b], sem_di[st][b])

    def _wait_idx(st, b):
        pltpu.make_async_copy(src_hbm.at[pl.ds(0, CHUNK)], sidx[st][b],
                              sem_si[st][b]).wait()
        pltpu.make_async_copy(dst_hbm.at[pl.ds(0, CHUNK)], didx[st][b],
                              sem_di[st][b]).wait()

    for b in range(NBUF):
        _start_idx(0, b, b)

    def _group(g, st, nst):
        q0 = g * NBUF
        for b in range(NBUF):
            @pl.when(jnp.logical_and(q0 + b - NBUF >= 0, q0 + b - NBUF < n_my))
            def _(b=b, nst=nst):
                pltpu.make_async_copy(rows[b], acc.at[didx[nst][b]],
                                      sem_a[b]).wait()

            @pl.when(q0 + b < n_my)
            def _(b=b, st=st):
                _wait_idx(st, b)
                pltpu.async_copy(table_hbm.at[sidx[st][b]], rows[b], sem_g[b])

            @pl.when(q0 + NBUF + b < n_my)
            def _(b=b, nst=nst):
                _start_idx(nst, b, q0 + NBUF + b)
        for b in range(NBUF):
            @pl.when(q0 + b < n_my)
            def _(b=b, st=st):
                pltpu.make_async_copy(table_hbm.at[sidx[st][b]], rows[b],
                                      sem_g[b]).wait()
                pltpu.async_copy(rows[b], acc.at[didx[st][b]], sem_a[b], add=True)

    @pl.loop(0, NGROUPS // 2)
    def _(gg):
        _group(2 * gg, 0, 1)
        _group(2 * gg + 1, 1, 0)

    plsc.subcore_barrier()

    def _wout(r0, nr):
        pltpu.sync_copy(acc.at[pl.ds(r0, nr)], out_hbm.at[c, pl.ds(r0, nr)])

    _striped(s, _wout)


@functools.partial(
    pl.kernel,
    out_type=jax.ShapeDtypeStruct((2048, D), jnp.float32),
    mesh=_mesh,
    scratch_types=[
        pltpu.VMEM((USERS_PER_W,), jnp.int32),
        pltpu.VMEM((USERS_PER_W, D), jnp.float32),
        pltpu.SemaphoreType.DMA,
    ],
)
def _user_gather_kernel(h_hbm, users_hbm, out_hbm, uidx, rows, sem):
    w = _worker_id()
    base = w * USERS_PER_W
    pltpu.sync_copy(users_hbm.at[pl.ds(base, USERS_PER_W)], uidx)
    pltpu.async_copy(h_hbm.at[uidx], rows, sem).wait()
    pltpu.sync_copy(rows, out_hbm.at[pl.ds(base, USERS_PER_W)])


_BLK = 1000


def _mm_body(x_ref, w_ref, o_ref):
    o_ref[...] = jnp.dot(x_ref[...], w_ref[...],
                         preferred_element_type=jnp.float32)


def _mm(x, w):
    n = x.shape[0]
    return pl.pallas_call(
        _mm_body,
        grid=(n // _BLK,),
        in_specs=[
            pl.BlockSpec((_BLK, x.shape[1]), lambda i: (i, 0)),
            pl.BlockSpec(w.shape, lambda i: (0, 0)),
        ],
        out_specs=pl.BlockSpec((_BLK, w.shape[1]), lambda i: (i, 0)),
        out_shape=jax.ShapeDtypeStruct((n, w.shape[1]), jnp.float32),
    )(x, w)


def _norm_from(deg_ref):
    d = deg_ref[0, :, 0:1] + deg_ref[1, :, 0:1]
    return lax.rsqrt(jnp.maximum(d, 1.0))


_DEG_SPEC = pl.BlockSpec((NC, _BLK, 16), lambda i: (0, i, 0))


def _scale_body(hw_ref, degs_ref, o_ref):
    o_ref[...] = hw_ref[...] * _norm_from(degs_ref)


def _scale(hw, degs_p):
    return pl.pallas_call(
        _scale_body,
        grid=(N // _BLK,),
        in_specs=[
            pl.BlockSpec((_BLK, D), lambda i: (i, 0)),
            _DEG_SPEC,
        ],
        out_specs=pl.BlockSpec((_BLK, D), lambda i: (i, 0)),
        out_shape=jax.ShapeDtypeStruct((N, D), jnp.float32),
    )(hw, degs_p)


def _layer_mm_body(p_ref, degd_ref, degs_ref, b_ref, w_ref, o_ref):
    agg = p_ref[0] + p_ref[1]
    h = jax.nn.relu(agg * _norm_from(degd_ref) + b_ref[...])
    o_ref[...] = jnp.dot(h, w_ref[...],
                         preferred_element_type=jnp.float32) * _norm_from(degs_ref)


def _layer_mm(p, degd_p, degs_p, b, w):
    return pl.pallas_call(
        _layer_mm_body,
        grid=(N // _BLK,),
        in_specs=[
            pl.BlockSpec((NC, _BLK, D), lambda i: (0, i, 0)),
            _DEG_SPEC,
            _DEG_SPEC,
            pl.BlockSpec((1, D), lambda i: (0, 0)),
            pl.BlockSpec((D, D), lambda i: (0, 0)),
        ],
        out_specs=pl.BlockSpec((_BLK, D), lambda i: (i, 0)),
        out_shape=jax.ShapeDtypeStruct((N, D), jnp.float32),
    )(p, degd_p, degs_p, b, w)


def _layer_out_body(p_ref, degd_ref, b_ref, o_ref):
    agg = p_ref[0] + p_ref[1]
    o_ref[...] = jax.nn.relu(agg * _norm_from(degd_ref) + b_ref[...])


def _layer_out(p, degd_p, b):
    return pl.pallas_call(
        _layer_out_body,
        grid=(N // _BLK,),
        in_specs=[
            pl.BlockSpec((NC, _BLK, D), lambda i: (0, i, 0)),
            _DEG_SPEC,
            pl.BlockSpec((1, D), lambda i: (0, 0)),
        ],
        out_specs=pl.BlockSpec((_BLK, D), lambda i: (i, 0)),
        out_shape=jax.ShapeDtypeStruct((N, D), jnp.float32),
    )(p, degd_p, b)


def _mlp_body(uh_ref, w1_ref, b1_ref, w2_ref, b2_ref, o_ref):
    t = jnp.tanh(jnp.dot(uh_ref[...], w1_ref[...],
                         preferred_element_type=jnp.float32) + b1_ref[...])
    o_ref[...] = jnp.dot(t, w2_ref[...],
                         preferred_element_type=jnp.float32) + b2_ref[...]


def _mlp(uh, w1, b1, w2, b2):
    return pl.pallas_call(
        _mlp_body,
        out_shape=jax.ShapeDtypeStruct((uh.shape[0], w2.shape[1]), jnp.float32),
    )(uh, w1, b1, w2, b2)


def kernel(features, W1, b1, W2, b2, Ws1, bs1, Ws2, bs2, edge_index, users):
    src = edge_index[0].astype(jnp.int32)
    dst = edge_index[1].astype(jnp.int32)
    users = users.astype(jnp.int32)
    zerosd = jnp.zeros((N, D), jnp.float32)
    zeros16 = jnp.zeros((N, 16), jnp.float32)
    ones16 = jnp.ones((CHUNK, 16), jnp.float32)

    degs_p, degd_p = _hist_kernel(src, dst, zeros16, ones16)
    hw1 = _mm(features, W1)
    scaled1 = _scale(hw1, degs_p)
    p1 = _scatter_kernel(scaled1, src, dst, zerosd)
    scaled2 = _layer_mm(p1, degd_p, degs_p, b1.reshape(1, D), W2)
    p2 = _scatter_kernel(scaled2, src, dst, zerosd)
    h = _layer_out(p2, degd_p, b2.reshape(1, D))
    uh = _user_gather_kernel(h, users)
    R = _mlp(uh, Ws1, bs1.reshape(1, -1), Ws2, bs2.reshape(1, -1))
    return (R, h)
